# baseline (speedup 1.0000x reference)
import functools

import jax
import jax.numpy as jnp
from jax import lax
from jax.experimental import pallas as pl
from jax.experimental.pallas import tpu as pltpu
from jax.experimental.pallas import tpu_sc as plsc

N = 10000
E = 320000
D = 128

NC = 2
NS = 16
L = 16
NW = NC * NS

CHUNK = 128
CHUNKS_TOTAL = E // CHUNK
NCH = CHUNKS_TOTAL // NW
NSTG = 3
NCHS = NCH // NSTG
NPAIR = NCHS // 2
EPTS = NCHS * CHUNK
XTRA = CHUNKS_TOTAL - NCH * NW
RBLK = 80
NRBLK = N // RBLK


def _matmul_body(x_ref, w_ref, b_ref, o_ref):
    o_ref[...] = lax.dot_general(
        x_ref[...], w_ref[...], (((1,), (1,)), ((), ())),
        preferred_element_type=jnp.float32,
    ) + b_ref[...]


def _matmul(x, W, b2d):
    grid = 10
    bm = N // grid
    return pl.pallas_call(
        _matmul_body,
        grid=(grid,),
        in_specs=[
            pl.BlockSpec((bm, D), lambda i: (i, 0)),
            pl.BlockSpec((D, D), lambda i: (0, 0)),
            pl.BlockSpec((1, D), lambda i: (0, 0)),
        ],
        out_specs=pl.BlockSpec((bm, D), lambda i: (i, 0)),
        out_shape=jax.ShapeDtypeStruct((N, D), jnp.float32),
    )(x, W, b2d)


def _scale_chunk(rows, b, val_v, voff):
    @plsc.parallel_loop(0, CHUNK // L)
    def _sgrp(g):
        vals = val_v[pl.ds(voff + g * L, L)]
        for ii in range(L):
            vb = lax.broadcast_in_dim(vals[ii], (L,), ())
            i = g * L + ii
            for jj in range(D // L):
                rows[b, i, pl.ds(jj * L, L)] = rows[b, i, pl.ds(jj * L, L)] * vb


def _edge_body(src_hbm, dst2_hbm, val_hbm, xw_hbm, out_hbm,
               acc, src_v, dst_v, val_v, rows, gsem, ssem):
    cid = lax.axis_index("c")
    sid = lax.axis_index("s")
    wid = sid * NC + cid
    e0 = wid * NCH * CHUNK
    c0 = wid * NCH

    def _gather(j, b):
        pltpu.async_copy(
            xw_hbm.at[src_v.at[pl.ds(j * CHUNK, CHUNK)]], rows.at[b], gsem)

    def _gather_wait(j, b):
        pltpu.make_async_copy(
            xw_hbm.at[src_v.at[pl.ds(j * CHUNK, CHUNK)]], rows.at[b],
            gsem).wait()

    def _scatter(j, b):
        pltpu.async_copy(rows.at[b], acc.at[dst_v.at[j, 0]], ssem, add=True)

    def _scatter_wait(j, b):
        pltpu.make_async_copy(rows.at[b], acc.at[dst_v.at[j, 0]], ssem).wait()

    def _zrow(i, carry):
        for bb in range(2):
            for j in range(D // L):
                rows[bb, i, pl.ds(j * L, L)] = jnp.zeros((L,), jnp.float32)
        return carry

    lax.fori_loop(0, CHUNK, _zrow, 0)

    base_rblk = NRBLK // NS
    nrblk = jnp.where(sid < NRBLK % NS, base_rblk + 1, base_rblk)

    def _zblk(k, carry):
        pltpu.sync_copy(rows.at[0].at[pl.ds(0, RBLK)],
                        acc.at[pl.ds((sid + k * NS) * RBLK, RBLK)])
        return carry

    lax.fori_loop(0, nrblk, _zblk, 0)
    plsc.subcore_barrier()

    for h in range(NSTG):
        pltpu.sync_copy(src_hbm.at[pl.ds(e0 + h * EPTS, EPTS)], src_v)
        pltpu.sync_copy(val_hbm.at[pl.ds(e0 + h * EPTS, EPTS)], val_v)
        pltpu.sync_copy(dst2_hbm.at[pl.ds(c0 + h * NCHS, NCHS)], dst_v)

        _gather(0, 0)

        def _pair(p, carry):
            j0 = 2 * p
            j1 = j0 + 1

            _gather_wait(j0, 0)

            @pl.when(p > 0)
            def _():
                _scatter_wait(j0 - 1, 1)

            _gather(j1, 1)
            _scale_chunk(rows, 0, val_v, j0 * CHUNK)
            _scatter(j0, 0)

            _gather_wait(j1, 1)
            _scale_chunk(rows, 1, val_v, j1 * CHUNK)
            _scatter_wait(j0, 0)

            _scatter(j1, 1)

            @pl.when(j0 + 2 < NCHS)
            def _():
                _gather(j0 + 2, 0)

            return carry

        lax.fori_loop(0, NPAIR, _pair, 0)
        _scatter_wait(NCHS - 1, 1)

    @pl.when(wid < XTRA)
    def _():
        ce = NCH * NW + wid
        eb = ce * CHUNK
        pltpu.sync_copy(src_hbm.at[pl.ds(eb, CHUNK)],
                        src_v.at[pl.ds(0, CHUNK)])
        pltpu.sync_copy(val_hbm.at[pl.ds(eb, CHUNK)],
                        val_v.at[pl.ds(0, CHUNK)])
        pltpu.sync_copy(dst2_hbm.at[pl.ds(ce, 1)], dst_v.at[pl.ds(0, 1)])
        _gather(0, 0)
        _gather_wait(0, 0)
        _scale_chunk(rows, 0, val_v, 0)
        pltpu.sync_copy(rows.at[0], acc.at[dst_v.at[0, 0]], add=True)

    plsc.subcore_barrier()

    def _cblk(k, carry):
        row0 = (sid + k * NS) * RBLK
        pltpu.sync_copy(acc.at[pl.ds(row0, RBLK)],
                        out_hbm.at[cid, pl.ds(row0, RBLK)])
        return carry

    lax.fori_loop(0, nrblk, _cblk, 0)


_edge_pass = functools.partial(
    pl.kernel,
    out_type=jax.ShapeDtypeStruct((NC, N, D), jnp.float32),
    mesh=plsc.VectorSubcoreMesh(
        core_axis_name="c", subcore_axis_name="s",
        num_cores=NC, num_subcores=NS,
    ),
    scratch_types=[
        pltpu.VMEM_SHARED((N, D), jnp.float32),
        pltpu.VMEM((EPTS,), jnp.int32),
        pltpu.VMEM((NCHS, 1, CHUNK), jnp.int32),
        pltpu.VMEM((EPTS,), jnp.float32),
        pltpu.VMEM((2, CHUNK, D), jnp.float32),
        pltpu.SemaphoreType.DMA,
        pltpu.SemaphoreType.DMA,
    ],
)(_edge_body)


def _combine_body(p_ref, o_ref):
    o_ref[...] = jnp.maximum(p_ref[0] + p_ref[1], 0.0)


def _combine(partial):
    grid = 10
    bm = N // grid
    return pl.pallas_call(
        _combine_body,
        grid=(grid,),
        in_specs=[pl.BlockSpec((NC, bm, D), lambda i: (0, i, 0))],
        out_specs=pl.BlockSpec((bm, D), lambda i: (i, 0)),
        out_shape=jax.ShapeDtypeStruct((N, D), jnp.float32),
    )(partial)


def kernel(edge_index, edge_values, x, W, b):
    xw = _matmul(x, W, b.reshape(1, D))
    dst2 = edge_index[0].reshape(CHUNKS_TOTAL, 1, CHUNK)
    partial = _edge_pass(edge_index[1], dst2, edge_values, xw)
    return _combine(partial)


# speedup vs baseline: 1.0102x; 1.0102x over previous
import functools

import jax
import jax.numpy as jnp
from jax import lax
from jax.experimental import pallas as pl
from jax.experimental.pallas import tpu as pltpu
from jax.experimental.pallas import tpu_sc as plsc

N = 10000
E = 320000
D = 128

NC = 2
NS = 16
L = 16
NW = NC * NS

CHUNK = 128
CHUNKS_TOTAL = E // CHUNK
NCH = CHUNKS_TOTAL // NW
NSTG = 3
NCHS = NCH // NSTG
NPAIR = NCHS // 2
EPTS = NCHS * CHUNK
XTRA = CHUNKS_TOTAL - NCH * NW
RBLK = 80
NRBLK = N // RBLK


def _matmul_body(x_ref, w_ref, b_ref, o_ref):
    o_ref[...] = lax.dot_general(
        x_ref[...], w_ref[...], (((1,), (1,)), ((), ())),
        preferred_element_type=jnp.float32,
    ) + b_ref[...]


def _matmul(x, W, b2d):
    grid = 10
    bm = N // grid
    return pl.pallas_call(
        _matmul_body,
        grid=(grid,),
        in_specs=[
            pl.BlockSpec((bm, D), lambda i: (i, 0)),
            pl.BlockSpec((D, D), lambda i: (0, 0)),
            pl.BlockSpec((1, D), lambda i: (0, 0)),
        ],
        out_specs=pl.BlockSpec((bm, D), lambda i: (i, 0)),
        out_shape=jax.ShapeDtypeStruct((N, D), jnp.float32),
    )(x, W, b2d)


def _scale_chunk(rows, b, val_v, voff):
    def _sgrp(g, carry):
        vals = val_v[pl.ds(voff + g * L, L)]
        for ii in range(L):
            vb = lax.broadcast_in_dim(vals[ii], (L,), ())
            i = g * L + ii
            for jj in range(D // L):
                rows[b, i, pl.ds(jj * L, L)] = rows[b, i, pl.ds(jj * L, L)] * vb
        return carry

    lax.fori_loop(0, CHUNK // L, _sgrp, 0)


def _edge_body(src_hbm, dst2_hbm, val_hbm, xw_hbm, out_hbm,
               acc, src_v, dst_v, val_v, rows, gsem, ssem):
    cid = lax.axis_index("c")
    sid = lax.axis_index("s")
    wid = sid * NC + cid
    e0 = wid * NCH * CHUNK
    c0 = wid * NCH

    def _gather(j, b):
        pltpu.async_copy(
            xw_hbm.at[src_v.at[pl.ds(j * CHUNK, CHUNK)]], rows.at[b], gsem)

    def _gather_wait(j, b):
        pltpu.make_async_copy(
            xw_hbm.at[src_v.at[pl.ds(j * CHUNK, CHUNK)]], rows.at[b],
            gsem).wait()

    def _scatter(j, b):
        pltpu.async_copy(rows.at[b], acc.at[dst_v.at[j, 0]], ssem, add=True)

    def _scatter_wait(j, b):
        pltpu.make_async_copy(rows.at[b], acc.at[dst_v.at[j, 0]], ssem).wait()

    def _zrow(i, carry):
        for bb in range(2):
            for j in range(D // L):
                rows[bb, i, pl.ds(j * L, L)] = jnp.zeros((L,), jnp.float32)
        return carry

    lax.fori_loop(0, CHUNK, _zrow, 0)

    base_rblk = NRBLK // NS
    nrblk = jnp.where(sid < NRBLK % NS, base_rblk + 1, base_rblk)

    def _zblk(k, carry):
        pltpu.sync_copy(rows.at[0].at[pl.ds(0, RBLK)],
                        acc.at[pl.ds((sid + k * NS) * RBLK, RBLK)])
        return carry

    lax.fori_loop(0, nrblk, _zblk, 0)
    plsc.subcore_barrier()

    for h in range(NSTG):
        pltpu.sync_copy(src_hbm.at[pl.ds(e0 + h * EPTS, EPTS)], src_v)
        pltpu.sync_copy(val_hbm.at[pl.ds(e0 + h * EPTS, EPTS)], val_v)
        pltpu.sync_copy(dst2_hbm.at[pl.ds(c0 + h * NCHS, NCHS)], dst_v)

        _gather(0, 0)

        def _pair(p, carry):
            j0 = 2 * p
            j1 = j0 + 1

            _gather_wait(j0, 0)

            @pl.when(p > 0)
            def _():
                _scatter_wait(j0 - 1, 1)

            _gather(j1, 1)
            _scale_chunk(rows, 0, val_v, j0 * CHUNK)
            _scatter(j0, 0)

            _gather_wait(j1, 1)
            _scale_chunk(rows, 1, val_v, j1 * CHUNK)
            _scatter_wait(j0, 0)

            _scatter(j1, 1)

            @pl.when(j0 + 2 < NCHS)
            def _():
                _gather(j0 + 2, 0)

            return carry

        lax.fori_loop(0, NPAIR, _pair, 0)
        _scatter_wait(NCHS - 1, 1)

    @pl.when(wid < XTRA)
    def _():
        ce = NCH * NW + wid
        eb = ce * CHUNK
        pltpu.sync_copy(src_hbm.at[pl.ds(eb, CHUNK)],
                        src_v.at[pl.ds(0, CHUNK)])
        pltpu.sync_copy(val_hbm.at[pl.ds(eb, CHUNK)],
                        val_v.at[pl.ds(0, CHUNK)])
        pltpu.sync_copy(dst2_hbm.at[pl.ds(ce, 1)], dst_v.at[pl.ds(0, 1)])
        _gather(0, 0)
        _gather_wait(0, 0)
        _scale_chunk(rows, 0, val_v, 0)
        pltpu.sync_copy(rows.at[0], acc.at[dst_v.at[0, 0]], add=True)

    plsc.subcore_barrier()

    def _cblk(k, carry):
        row0 = (sid + k * NS) * RBLK
        pltpu.sync_copy(acc.at[pl.ds(row0, RBLK)],
                        out_hbm.at[cid, pl.ds(row0, RBLK)])
        return carry

    lax.fori_loop(0, nrblk, _cblk, 0)


_edge_pass = functools.partial(
    pl.kernel,
    out_type=jax.ShapeDtypeStruct((NC, N, D), jnp.float32),
    mesh=plsc.VectorSubcoreMesh(
        core_axis_name="c", subcore_axis_name="s",
        num_cores=NC, num_subcores=NS,
    ),
    scratch_types=[
        pltpu.VMEM_SHARED((N, D), jnp.float32),
        pltpu.VMEM((EPTS,), jnp.int32),
        pltpu.VMEM((NCHS, 1, CHUNK), jnp.int32),
        pltpu.VMEM((EPTS,), jnp.float32),
        pltpu.VMEM((2, CHUNK, D), jnp.float32),
        pltpu.SemaphoreType.DMA,
        pltpu.SemaphoreType.DMA,
    ],
)(_edge_body)


def _combine_body(p_ref, o_ref):
    o_ref[...] = jnp.maximum(p_ref[0] + p_ref[1], 0.0)


def _combine(partial):
    grid = 10
    bm = N // grid
    return pl.pallas_call(
        _combine_body,
        grid=(grid,),
        in_specs=[pl.BlockSpec((NC, bm, D), lambda i: (0, i, 0))],
        out_specs=pl.BlockSpec((bm, D), lambda i: (i, 0)),
        out_shape=jax.ShapeDtypeStruct((N, D), jnp.float32),
    )(partial)


def kernel(edge_index, edge_values, x, W, b):
    xw = _matmul(x, W, b.reshape(1, D))
    dst2 = edge_index[0].reshape(CHUNKS_TOTAL, 1, CHUNK)
    partial = _edge_pass(edge_index[1], dst2, edge_values, xw)
    return _combine(partial)
